# Initial kernel scaffold; baseline (speedup 1.0000x reference)
import jax
import jax.numpy as jnp
from jax.experimental import pallas as pl


def kernel(t, state_H, state_K, ind_K, ind_HK, kappa_K, kappa_H, W1, b1, W2, b2, omega, w_hop):
    raise NotImplementedError("write your pallas kernel here")


# speedup vs baseline: 16.2004x; 16.2004x over previous
import functools

import jax
import jax.numpy as jnp
from jax import lax
from jax.experimental import pallas as pl
from jax.experimental.pallas import tpu as pltpu
from jax.experimental.pallas import tpu_sc as plsc

N = 100000
D = 16
H = 32
E_K = 3200000
E_HK = 1600000

NC = 2
NS = 16
NW = NC * NS
B = 128
T = 1024

NP = 100096
ROWS = NP // NS
EKP = 782 * B * NW
EHKP = 391 * B * NW

BN = 2000
GRID = N // BN

f32 = jnp.float32
i32 = jnp.int32


def _prep_body(sh_ref, sk_ref, wh_ref, kh_ref, skn_ref, g1_ref, fhb_ref):
  sh = sh_ref[...]
  g = jnp.tanh(sh)
  fhb_ref[...] = -sh + wh_ref[...] * g
  g1_ref[...] = g / kh_ref[0, 0]
  sk = sk_ref[...]
  nrm = lax.rsqrt(jnp.sum(sk * sk, axis=1, keepdims=True))
  skn_ref[...] = sk * nrm


_prep = pl.pallas_call(
    _prep_body,
    grid=(GRID,),
    in_specs=[
        pl.BlockSpec((BN, 1), lambda i: (i, 0)),
        pl.BlockSpec((BN, D), lambda i: (i, 0)),
        pl.BlockSpec((BN, 1), lambda i: (i, 0)),
        pl.BlockSpec((1, 1), lambda i: (0, 0)),
    ],
    out_specs=[
        pl.BlockSpec((BN, D), lambda i: (i, 0)),
        pl.BlockSpec((BN, 1), lambda i: (i, 0)),
        pl.BlockSpec((BN, 1), lambda i: (i, 0)),
    ],
    out_shape=[
        jax.ShapeDtypeStruct((N, D), f32),
        jax.ShapeDtypeStruct((N, 1), f32),
        jax.ShapeDtypeStruct((N, 1), f32),
    ],
)


def _tbl_body(w1_ref, b1_ref, w2_ref, b2_ref, tbl_ref):
  w1 = w1_ref[...]
  b1 = b1_ref[...]
  w2 = w2_ref[...]
  r = lax.broadcasted_iota(i32, (8, 128), 0)
  c = lax.broadcasted_iota(i32, (8, 128), 1)
  x = (r * 128 + c).astype(f32) * (2.0 / (T - 1)) - 1.0
  acc = jnp.full((8, 128), b2_ref[0, 0], f32)
  for h in range(H):
    acc = acc + w2[h, 0] * jnp.tanh(x * w1[0, h] + b1[0, h])
  tbl_ref[...] = acc


_tbl = pl.pallas_call(
    _tbl_body,
    out_shape=jax.ShapeDtypeStruct((8, 128), f32),
)


def _post_body(skn_ref, aka_ref, akb_ref, aha_ref, ahb_ref, fhb_ref, om_ref,
               fh_ref, fk_ref):
  skn = skn_ref[...]
  fk = aka_ref[...] + akb_ref[...]
  om = om_ref[...]
  a = (om - om.T) * 0.5
  fk_ref[...] = (-fk + skn * jnp.sum(skn * fk, axis=1, keepdims=True)
                 + jnp.dot(skn, a, preferred_element_type=f32))
  fh_ref[...] = fhb_ref[...] + aha_ref[...] + ahb_ref[...]


_post = pl.pallas_call(
    _post_body,
    grid=(GRID,),
    in_specs=[
        pl.BlockSpec((BN, D), lambda i: (i, 0)),
        pl.BlockSpec((BN, D), lambda i: (i, 0)),
        pl.BlockSpec((BN, D), lambda i: (i, 0)),
        pl.BlockSpec((BN, 1), lambda i: (i, 0)),
        pl.BlockSpec((BN, 1), lambda i: (i, 0)),
        pl.BlockSpec((BN, 1), lambda i: (i, 0)),
        pl.BlockSpec((D, D), lambda i: (0, 0)),
    ],
    out_specs=[
        pl.BlockSpec((BN, 1), lambda i: (i, 0)),
        pl.BlockSpec((BN, D), lambda i: (i, 0)),
    ],
    out_shape=[
        jax.ShapeDtypeStruct((N, 1), f32),
        jax.ShapeDtypeStruct((N, D), f32),
    ],
)


def _make_sc_edges():
  mesh = plsc.VectorSubcoreMesh(core_axis_name="c", subcore_axis_name="s")

  @functools.partial(
      pl.kernel,
      mesh=mesh,
      compiler_params=pltpu.CompilerParams(
          needs_layout_passes=False, use_tc_tiling_on_sc=False),
      out_type=[
          jax.ShapeDtypeStruct((NC * NP, D), f32),
          jax.ShapeDtypeStruct((NC * NP,), f32),
      ],
      scratch_types=[
          pltpu.VMEM_SHARED((NP, D), f32),
          pltpu.VMEM_SHARED((NP,), f32),
          pltpu.VMEM((T,), f32),
          pltpu.VMEM((16,), f32),
          pltpu.VMEM((B,), i32),
          pltpu.VMEM((B,), i32),
          pltpu.VMEM((B, D), f32),
          pltpu.VMEM((B, D), f32),
          pltpu.VMEM((B,), f32),
          pltpu.VMEM((B,), f32),
          pltpu.VMEM((B, D), f32),
          pltpu.VMEM((B, D), f32),
          pltpu.VMEM((B,), f32),
          pltpu.VMEM((B,), f32),
          pltpu.VMEM((16 * D,), f32),
          pltpu.SemaphoreType.DMA,
          pltpu.SemaphoreType.DMA,
      ],
  )
  def _sc_edges(skn_hbm, g1_hbm, ak_hbm, bk_hbm, ahk_hbm, bhk_hbm, tbl_hbm,
                kv_hbm, zk_hbm, zh_hbm, outk_hbm, outh_hbm,
                acck, acch, tbl_v, kv, aidx, bidx, xa, xb, g1a, g1b,
                msga, msgb, fha, fhb, prods, gsem, ssem):
    cid = lax.axis_index("c")
    sid = lax.axis_index("s")
    wid = sid * NC + cid

    pltpu.sync_copy(tbl_hbm, tbl_v)
    pltpu.sync_copy(kv_hbm, kv)
    r0 = sid * ROWS
    pltpu.sync_copy(zk_hbm.at[pl.ds(r0, ROWS)], acck.at[pl.ds(r0, ROWS)])
    pltpu.sync_copy(zh_hbm.at[pl.ds(r0, ROWS)], acch.at[pl.ds(r0, ROWS)])
    plsc.subcore_barrier()

    lane = lax.iota(i32, 16)

    def group_dot(j):
      e0 = j * 16
      va = []
      vb = []
      for e in range(16):
        a_row = xa[e0 + e, :]
        b_row = xb[e0 + e, :]
        va.append(a_row)
        vb.append(b_row)
        prods[pl.ds(e * D, D)] = a_row * b_row
      s = plsc.load_gather(prods, [lane * D])
      for d in range(1, D):
        s = s + plsc.load_gather(prods, [lane * D + d])
      return va, vb, s

    ek0 = wid * (EKP // NW)

    def k_batch(ib, carry):
      base = ek0 + ib * B
      pltpu.sync_copy(ak_hbm.at[pl.ds(base, B)], aidx)
      pltpu.sync_copy(bk_hbm.at[pl.ds(base, B)], bidx)
      cpa = pltpu.async_copy(skn_hbm.at[aidx], xa, gsem)
      cpb = pltpu.async_copy(skn_hbm.at[bidx], xb, gsem)
      cpa.wait()
      cpb.wait()

      def group(j, c2):
        va, vb, s = group_dot(j)
        q = jnp.clip((s + 1.0) * ((T - 1) * 0.5), 0.0, T - 1.0)
        ii = jnp.minimum(q.astype(i32), T - 2)
        v0 = plsc.load_gather(tbl_v, [ii])
        v1 = plsc.load_gather(tbl_v, [ii + 1])
        c = v0 + (q - ii.astype(f32)) * (v1 - v0)
        e0 = j * 16
        for e in range(16):
          ce = c[e]
          msga[e0 + e, :] = ce * vb[e]
          msgb[e0 + e, :] = ce * va[e]
        return c2

      lax.fori_loop(0, B // 16, group, 0)
      sca = pltpu.async_copy(msga, acck.at[aidx], ssem, add=True)
      scb = pltpu.async_copy(msgb, acck.at[bidx], ssem, add=True)
      sca.wait()
      scb.wait()
      return carry

    lax.fori_loop(0, EKP // (NW * B), k_batch, 0)

    eh0 = wid * (EHKP // NW)

    def hk_batch(ib, carry):
      base = eh0 + ib * B
      pltpu.sync_copy(ahk_hbm.at[pl.ds(base, B)], aidx)
      pltpu.sync_copy(bhk_hbm.at[pl.ds(base, B)], bidx)
      cpa = pltpu.async_copy(skn_hbm.at[aidx], xa, gsem)
      cpb = pltpu.async_copy(skn_hbm.at[bidx], xb, gsem)
      cga = pltpu.async_copy(g1_hbm.at[aidx], g1a, gsem)
      cgb = pltpu.async_copy(g1_hbm.at[bidx], g1b, gsem)
      cpa.wait()
      cpb.wait()
      cga.wait()
      cgb.wait()

      def group(j, c2):
        va, vb, gram = group_dot(j)
        e0 = j * 16
        ga = g1a[pl.ds(e0, 16)]
        gb = g1b[pl.ds(e0, 16)]
        fha[pl.ds(e0, 16)] = gram * gb
        fhb[pl.ds(e0, 16)] = gram * ga
        negg = kv[...] * ga * gb
        for e in range(16):
          ge = negg[e]
          msga[e0 + e, :] = ge * vb[e]
          msgb[e0 + e, :] = ge * va[e]
        return c2

      lax.fori_loop(0, B // 16, group, 0)
      sca = pltpu.async_copy(msga, acck.at[aidx], ssem, add=True)
      scb = pltpu.async_copy(msgb, acck.at[bidx], ssem, add=True)
      sha = pltpu.async_copy(fha, acch.at[aidx], ssem, add=True)
      shb = pltpu.async_copy(fhb, acch.at[bidx], ssem, add=True)
      sca.wait()
      scb.wait()
      sha.wait()
      shb.wait()
      return carry

    lax.fori_loop(0, EHKP // (NW * B), hk_batch, 0)

    plsc.subcore_barrier()
    pltpu.sync_copy(acck.at[pl.ds(r0, ROWS)],
                    outk_hbm.at[pl.ds(cid * NP + r0, ROWS)])
    pltpu.sync_copy(acch.at[pl.ds(r0, ROWS)],
                    outh_hbm.at[pl.ds(cid * NP + r0, ROWS)])

  return _sc_edges


_sc_edges = _make_sc_edges()


def kernel(t, state_H, state_K, ind_K, ind_HK, kappa_K, kappa_H, W1, b1, W2,
           b2, omega, w_hop):
  del t
  sh2 = state_H.reshape(N, 1).astype(f32)
  wh2 = w_hop.reshape(N, 1).astype(f32)
  kh = jnp.reshape(kappa_H, (1, 1)).astype(f32)
  kk = jnp.reshape(kappa_K, (1, 1)).astype(f32)

  skn, g1, fhb = _prep(sh2, state_K.astype(f32), wh2, kh)
  tbl = _tbl(W1.astype(f32), b1.reshape(1, H).astype(f32),
             W2.astype(f32), b2.reshape(1, 1).astype(f32)).reshape(T)

  kvec = jnp.broadcast_to(
      (-(kappa_H.astype(f32) ** 2) / kappa_K.astype(f32)).reshape(1), (16,))

  sknp = jnp.concatenate([skn, jnp.zeros((NP - N, D), f32)], axis=0)
  g1p = jnp.concatenate([g1.reshape(N), jnp.zeros((NP - N,), f32)], axis=0)

  pad_k = jnp.full((EKP - E_K,), N, i32)
  pad_hk = jnp.full((EHKP - E_HK,), N, i32)
  ak = jnp.concatenate([ind_K[:, 0].astype(i32), pad_k])
  bk = jnp.concatenate([ind_K[:, 1].astype(i32), pad_k])
  ahk = jnp.concatenate([ind_HK[:, 0].astype(i32), pad_hk])
  bhk = jnp.concatenate([ind_HK[:, 1].astype(i32), pad_hk])

  zk = jnp.zeros((NP, D), f32)
  zh = jnp.zeros((NP,), f32)

  outk, outh = _sc_edges(sknp, g1p, ak, bk, ahk, bhk, tbl, kvec, zk, zh)

  aka = outk[:N]
  akb = outk[NP:NP + N]
  aha = outh[:N].reshape(N, 1)
  ahb = outh[NP:NP + N].reshape(N, 1)

  fh, fk = _post(skn, aka, akb, aha, ahb, fhb, omega.astype(f32))
  return fh.reshape(N), fk


# speedup vs baseline: 20.7380x; 1.2801x over previous
import functools

import jax
import jax.numpy as jnp
from jax import lax
from jax.experimental import pallas as pl
from jax.experimental.pallas import tpu as pltpu
from jax.experimental.pallas import tpu_sc as plsc

N = 100000
D = 16
H = 32
E_K = 3200000
E_HK = 1600000

NC = 2
NS = 16
NW = NC * NS
B = 128
T = 1024

NP = 100096
ROWS = NP // NS
W = 2
EKP = 392 * W * B * NW
EHKP = 196 * W * B * NW

BN = 2000
GRID = N // BN

f32 = jnp.float32
i32 = jnp.int32


def _prep_body(sh_ref, sk_ref, wh_ref, kh_ref, skn_ref, g1_ref, fhb_ref):
  sh = sh_ref[...]
  g = jnp.tanh(sh)
  fhb_ref[...] = -sh + wh_ref[...] * g
  g1_ref[...] = g / kh_ref[0, 0]
  sk = sk_ref[...]
  nrm = lax.rsqrt(jnp.sum(sk * sk, axis=1, keepdims=True))
  skn_ref[...] = sk * nrm


_prep = pl.pallas_call(
    _prep_body,
    grid=(GRID,),
    in_specs=[
        pl.BlockSpec((BN, 1), lambda i: (i, 0)),
        pl.BlockSpec((BN, D), lambda i: (i, 0)),
        pl.BlockSpec((BN, 1), lambda i: (i, 0)),
        pl.BlockSpec((1, 1), lambda i: (0, 0)),
    ],
    out_specs=[
        pl.BlockSpec((BN, D), lambda i: (i, 0)),
        pl.BlockSpec((BN, 1), lambda i: (i, 0)),
        pl.BlockSpec((BN, 1), lambda i: (i, 0)),
    ],
    out_shape=[
        jax.ShapeDtypeStruct((N, D), f32),
        jax.ShapeDtypeStruct((N, 1), f32),
        jax.ShapeDtypeStruct((N, 1), f32),
    ],
)


def _tbl_body(w1_ref, b1_ref, w2_ref, b2_ref, tbl_ref):
  w1 = w1_ref[...]
  b1 = b1_ref[...]
  w2 = w2_ref[...]
  r = lax.broadcasted_iota(i32, (8, 128), 0)
  c = lax.broadcasted_iota(i32, (8, 128), 1)
  x = (r * 128 + c).astype(f32) * (2.0 / (T - 1)) - 1.0
  acc = jnp.full((8, 128), b2_ref[0, 0], f32)
  for h in range(H):
    acc = acc + w2[h, 0] * jnp.tanh(x * w1[0, h] + b1[0, h])
  tbl_ref[...] = acc


_tbl = pl.pallas_call(
    _tbl_body,
    out_shape=jax.ShapeDtypeStruct((8, 128), f32),
)


def _post_body(skn_ref, aka_ref, akb_ref, aha_ref, ahb_ref, fhb_ref, om_ref,
               fh_ref, fk_ref):
  skn = skn_ref[...]
  fk = aka_ref[...] + akb_ref[...]
  om = om_ref[...]
  a = (om - om.T) * 0.5
  fk_ref[...] = (-fk + skn * jnp.sum(skn * fk, axis=1, keepdims=True)
                 + jnp.dot(skn, a, preferred_element_type=f32))
  fh_ref[...] = fhb_ref[...] + aha_ref[...] + ahb_ref[...]


_post = pl.pallas_call(
    _post_body,
    grid=(GRID,),
    in_specs=[
        pl.BlockSpec((BN, D), lambda i: (i, 0)),
        pl.BlockSpec((BN, D), lambda i: (i, 0)),
        pl.BlockSpec((BN, D), lambda i: (i, 0)),
        pl.BlockSpec((BN, 1), lambda i: (i, 0)),
        pl.BlockSpec((BN, 1), lambda i: (i, 0)),
        pl.BlockSpec((BN, 1), lambda i: (i, 0)),
        pl.BlockSpec((D, D), lambda i: (0, 0)),
    ],
    out_specs=[
        pl.BlockSpec((BN, 1), lambda i: (i, 0)),
        pl.BlockSpec((BN, D), lambda i: (i, 0)),
    ],
    out_shape=[
        jax.ShapeDtypeStruct((N, 1), f32),
        jax.ShapeDtypeStruct((N, D), f32),
    ],
)


def _make_sc_edges():
  mesh = plsc.VectorSubcoreMesh(core_axis_name="c", subcore_axis_name="s")

  @functools.partial(
      pl.kernel,
      mesh=mesh,
      compiler_params=pltpu.CompilerParams(
          needs_layout_passes=False, use_tc_tiling_on_sc=False),
      out_type=[
          jax.ShapeDtypeStruct((NC * NP, D), f32),
          jax.ShapeDtypeStruct((NC * NP,), f32),
      ],
      scratch_types=[
          pltpu.VMEM_SHARED((NP, D), f32),
          pltpu.VMEM_SHARED((NP,), f32),
          pltpu.VMEM((T,), f32),
          pltpu.VMEM((16,), f32),
          pltpu.VMEM((W, B), i32),
          pltpu.VMEM((W, B), i32),
          [pltpu.VMEM((B, D), f32)] * W,
          [pltpu.VMEM((B, D), f32)] * W,
          [pltpu.VMEM((B,), f32)] * W,
          [pltpu.VMEM((B,), f32)] * W,
          [pltpu.VMEM((B, D), f32)] * W,
          [pltpu.VMEM((B, D), f32)] * W,
          [pltpu.VMEM((B,), f32)] * W,
          [pltpu.VMEM((B,), f32)] * W,
          pltpu.VMEM((16 * D,), f32),
          pltpu.SemaphoreType.DMA,
          pltpu.SemaphoreType.DMA,
          pltpu.SemaphoreType.DMA,
      ],
  )
  def _sc_edges(skn_hbm, g1_hbm, ak_hbm, bk_hbm, ahk_hbm, bhk_hbm, tbl_hbm,
                kv_hbm, zk_hbm, zh_hbm, outk_hbm, outh_hbm,
                acck, acch, tbl_v, kv, aidx, bidx, xa, xb, g1a, g1b,
                msga, msgb, fha, fhb, prods, isem, gsem, ssem):
    cid = lax.axis_index("c")
    sid = lax.axis_index("s")
    wid = sid * NC + cid

    pltpu.sync_copy(tbl_hbm, tbl_v)
    pltpu.sync_copy(kv_hbm, kv)
    r0 = sid * ROWS
    pltpu.sync_copy(zk_hbm.at[pl.ds(r0, ROWS)], acck.at[pl.ds(r0, ROWS)])
    pltpu.sync_copy(zh_hbm.at[pl.ds(r0, ROWS)], acch.at[pl.ds(r0, ROWS)])
    plsc.subcore_barrier()

    lane = lax.iota(i32, 16)

    def group_dot(xa_w, xb_w, j):
      e0 = j * 16
      va = []
      vb = []
      for e in range(16):
        a_row = xa_w[e0 + e, :]
        b_row = xb_w[e0 + e, :]
        va.append(a_row)
        vb.append(b_row)
        prods[pl.ds(e * D, D)] = a_row * b_row
      s = plsc.load_gather(prods, [lane * D])
      for d in range(1, D):
        s = s + plsc.load_gather(prods, [lane * D + d])
      return va, vb, s

    krow0 = wid * (EKP // (NW * B))

    def k_iter(it, carry):
      rb = krow0 + it * W
      ia = pltpu.async_copy(ak_hbm.at[pl.ds(rb, W)], aidx, isem)
      ib_ = pltpu.async_copy(bk_hbm.at[pl.ds(rb, W)], bidx, isem)
      ia.wait()
      ib_.wait()
      cps = []
      for w in range(W):
        cps.append(pltpu.async_copy(skn_hbm.at[aidx.at[w]], xa[w], gsem))
        cps.append(pltpu.async_copy(skn_hbm.at[bidx.at[w]], xb[w], gsem))
      for cp in cps:
        cp.wait()

      for w in range(W):
        def group(j, c2, w=w):
          va, vb, s = group_dot(xa[w], xb[w], j)
          q = jnp.clip((s + 1.0) * ((T - 1) * 0.5), 0.0, T - 1.0)
          ii = jnp.minimum(q.astype(i32), T - 2)
          v0 = plsc.load_gather(tbl_v, [ii])
          v1 = plsc.load_gather(tbl_v, [ii + 1])
          c = v0 + (q - ii.astype(f32)) * (v1 - v0)
          e0 = j * 16
          for e in range(16):
            ce = c[e]
            msga[w][e0 + e, :] = ce * vb[e]
            msgb[w][e0 + e, :] = ce * va[e]
          return c2

        lax.fori_loop(0, B // 16, group, 0)

      scs = []
      for w in range(W):
        scs.append(
            pltpu.async_copy(msga[w], acck.at[aidx.at[w]], ssem, add=True))
        scs.append(
            pltpu.async_copy(msgb[w], acck.at[bidx.at[w]], ssem, add=True))
      for sc in scs:
        sc.wait()
      return carry

    lax.fori_loop(0, EKP // (NW * B * W), k_iter, 0)

    hrow0 = wid * (EHKP // (NW * B))

    def hk_iter(it, carry):
      rb = hrow0 + it * W
      ia = pltpu.async_copy(ahk_hbm.at[pl.ds(rb, W)], aidx, isem)
      ib_ = pltpu.async_copy(bhk_hbm.at[pl.ds(rb, W)], bidx, isem)
      ia.wait()
      ib_.wait()
      cps = []
      for w in range(W):
        cps.append(pltpu.async_copy(skn_hbm.at[aidx.at[w]], xa[w], gsem))
        cps.append(pltpu.async_copy(skn_hbm.at[bidx.at[w]], xb[w], gsem))
        cps.append(pltpu.async_copy(g1_hbm.at[aidx.at[w]], g1a[w], gsem))
        cps.append(pltpu.async_copy(g1_hbm.at[bidx.at[w]], g1b[w], gsem))
      for cp in cps:
        cp.wait()

      for w in range(W):
        def group(j, c2, w=w):
          va, vb, gram = group_dot(xa[w], xb[w], j)
          e0 = j * 16
          ga = g1a[w][pl.ds(e0, 16)]
          gb = g1b[w][pl.ds(e0, 16)]
          fha[w][pl.ds(e0, 16)] = gram * gb
          fhb[w][pl.ds(e0, 16)] = gram * ga
          negg = kv[...] * ga * gb
          for e in range(16):
            ge = negg[e]
            msga[w][e0 + e, :] = ge * vb[e]
            msgb[w][e0 + e, :] = ge * va[e]
          return c2

        lax.fori_loop(0, B // 16, group, 0)

      scs = []
      for w in range(W):
        scs.append(
            pltpu.async_copy(msga[w], acck.at[aidx.at[w]], ssem, add=True))
        scs.append(
            pltpu.async_copy(msgb[w], acck.at[bidx.at[w]], ssem, add=True))
        scs.append(
            pltpu.async_copy(fha[w], acch.at[aidx.at[w]], ssem, add=True))
        scs.append(
            pltpu.async_copy(fhb[w], acch.at[bidx.at[w]], ssem, add=True))
      for sc in scs:
        sc.wait()
      return carry

    lax.fori_loop(0, EHKP // (NW * B * W), hk_iter, 0)

    plsc.subcore_barrier()
    pltpu.sync_copy(acck.at[pl.ds(r0, ROWS)],
                    outk_hbm.at[pl.ds(cid * NP + r0, ROWS)])
    pltpu.sync_copy(acch.at[pl.ds(r0, ROWS)],
                    outh_hbm.at[pl.ds(cid * NP + r0, ROWS)])

  return _sc_edges


_sc_edges = _make_sc_edges()


def kernel(t, state_H, state_K, ind_K, ind_HK, kappa_K, kappa_H, W1, b1, W2,
           b2, omega, w_hop):
  del t
  sh2 = state_H.reshape(N, 1).astype(f32)
  wh2 = w_hop.reshape(N, 1).astype(f32)
  kh = jnp.reshape(kappa_H, (1, 1)).astype(f32)
  kk = jnp.reshape(kappa_K, (1, 1)).astype(f32)

  skn, g1, fhb = _prep(sh2, state_K.astype(f32), wh2, kh)
  tbl = _tbl(W1.astype(f32), b1.reshape(1, H).astype(f32),
             W2.astype(f32), b2.reshape(1, 1).astype(f32)).reshape(T)

  kvec = jnp.broadcast_to(
      (-(kappa_H.astype(f32) ** 2) / kappa_K.astype(f32)).reshape(1), (16,))

  sknp = jnp.concatenate([skn, jnp.zeros((NP - N, D), f32)], axis=0)
  g1p = jnp.concatenate([g1.reshape(N), jnp.zeros((NP - N,), f32)], axis=0)

  pad_k = jnp.full((EKP - E_K,), N, i32)
  pad_hk = jnp.full((EHKP - E_HK,), N, i32)
  ak = jnp.concatenate([ind_K[:, 0].astype(i32), pad_k]).reshape(EKP // B, B)
  bk = jnp.concatenate([ind_K[:, 1].astype(i32), pad_k]).reshape(EKP // B, B)
  ahk = jnp.concatenate(
      [ind_HK[:, 0].astype(i32), pad_hk]).reshape(EHKP // B, B)
  bhk = jnp.concatenate(
      [ind_HK[:, 1].astype(i32), pad_hk]).reshape(EHKP // B, B)

  zk = jnp.zeros((NP, D), f32)
  zh = jnp.zeros((NP,), f32)

  outk, outh = _sc_edges(sknp, g1p, ak, bk, ahk, bhk, tbl, kvec, zk, zh)

  aka = outk[:N]
  akb = outk[NP:NP + N]
  aha = outh[:N].reshape(N, 1)
  ahb = outh[NP:NP + N].reshape(N, 1)

  fh, fk = _post(skn, aka, akb, aha, ahb, fhb, omega.astype(f32))
  return fh.reshape(N), fk


# speedup vs baseline: 31.4473x; 1.5164x over previous
import functools

import jax
import jax.numpy as jnp
from jax import lax
from jax.experimental import pallas as pl
from jax.experimental.pallas import tpu as pltpu
from jax.experimental.pallas import tpu_sc as plsc

N = 100000
D = 16
H = 32
E_K = 3200000
E_HK = 1600000

NC = 2
NS = 16
NW = NC * NS
B = 128
T = 1024

NP = 100096
ROWS = NP // NS
W = 2
EKP = 392 * W * B * NW
EHKP = 196 * W * B * NW

BN = 2000
GRID = N // BN

f32 = jnp.float32
i32 = jnp.int32


def _prep_body(sh_ref, sk_ref, wh_ref, kh_ref, skn_ref, g1_ref, fhb_ref):
  sh = sh_ref[...]
  g = jnp.tanh(sh)
  fhb_ref[...] = -sh + wh_ref[...] * g
  g1_ref[...] = g / kh_ref[0, 0]
  sk = sk_ref[...]
  nrm = lax.rsqrt(jnp.sum(sk * sk, axis=1, keepdims=True))
  skn_ref[...] = sk * nrm


_prep = pl.pallas_call(
    _prep_body,
    grid=(GRID,),
    in_specs=[
        pl.BlockSpec((BN, 1), lambda i: (i, 0)),
        pl.BlockSpec((BN, D), lambda i: (i, 0)),
        pl.BlockSpec((BN, 1), lambda i: (i, 0)),
        pl.BlockSpec((1, 1), lambda i: (0, 0)),
    ],
    out_specs=[
        pl.BlockSpec((BN, D), lambda i: (i, 0)),
        pl.BlockSpec((BN, 1), lambda i: (i, 0)),
        pl.BlockSpec((BN, 1), lambda i: (i, 0)),
    ],
    out_shape=[
        jax.ShapeDtypeStruct((N, D), f32),
        jax.ShapeDtypeStruct((N, 1), f32),
        jax.ShapeDtypeStruct((N, 1), f32),
    ],
)


def _tbl_body(w1_ref, b1_ref, w2_ref, b2_ref, tbl_ref):
  w1 = w1_ref[...]
  b1 = b1_ref[...]
  w2 = w2_ref[...]
  r = lax.broadcasted_iota(i32, (8, 128), 0)
  c = lax.broadcasted_iota(i32, (8, 128), 1)
  x = (r * 128 + c).astype(f32) * (2.0 / (T - 1)) - 1.0
  acc = jnp.full((8, 128), b2_ref[0, 0], f32)
  for h in range(H):
    acc = acc + w2[h, 0] * jnp.tanh(x * w1[0, h] + b1[0, h])
  tbl_ref[...] = acc


_tbl = pl.pallas_call(
    _tbl_body,
    out_shape=jax.ShapeDtypeStruct((8, 128), f32),
)


def _post_body(skn_ref, aka_ref, akb_ref, aha_ref, ahb_ref, fhb_ref, om_ref,
               fh_ref, fk_ref):
  skn = skn_ref[...]
  fk = aka_ref[...] + akb_ref[...]
  om = om_ref[...]
  a = (om - om.T) * 0.5
  fk_ref[...] = (-fk + skn * jnp.sum(skn * fk, axis=1, keepdims=True)
                 + jnp.dot(skn, a, preferred_element_type=f32))
  fh_ref[...] = fhb_ref[...] + aha_ref[...] + ahb_ref[...]


_post = pl.pallas_call(
    _post_body,
    grid=(GRID,),
    in_specs=[
        pl.BlockSpec((BN, D), lambda i: (i, 0)),
        pl.BlockSpec((BN, D), lambda i: (i, 0)),
        pl.BlockSpec((BN, D), lambda i: (i, 0)),
        pl.BlockSpec((BN, 1), lambda i: (i, 0)),
        pl.BlockSpec((BN, 1), lambda i: (i, 0)),
        pl.BlockSpec((BN, 1), lambda i: (i, 0)),
        pl.BlockSpec((D, D), lambda i: (0, 0)),
    ],
    out_specs=[
        pl.BlockSpec((BN, 1), lambda i: (i, 0)),
        pl.BlockSpec((BN, D), lambda i: (i, 0)),
    ],
    out_shape=[
        jax.ShapeDtypeStruct((N, 1), f32),
        jax.ShapeDtypeStruct((N, D), f32),
    ],
)


def _make_sc_edges():
  mesh = plsc.VectorSubcoreMesh(core_axis_name="c", subcore_axis_name="s")

  @functools.partial(
      pl.kernel,
      mesh=mesh,
      compiler_params=pltpu.CompilerParams(
          needs_layout_passes=False, use_tc_tiling_on_sc=False),
      out_type=[
          jax.ShapeDtypeStruct((NC * NP, D), f32),
          jax.ShapeDtypeStruct((NC * NP,), f32),
      ],
      scratch_types=[
          pltpu.VMEM_SHARED((NP, D), f32),
          pltpu.VMEM_SHARED((NP,), f32),
          pltpu.VMEM((T,), f32),
          pltpu.VMEM((16,), f32),
          pltpu.VMEM((4, B), i32),
          pltpu.VMEM((4, B), i32),
          [pltpu.VMEM((B, D), f32)] * 2,
          [pltpu.VMEM((B, D), f32)] * 2,
          [pltpu.VMEM((B,), f32)] * 2,
          [pltpu.VMEM((B,), f32)] * 2,
          [pltpu.VMEM((B, D), f32)] * 2,
          [pltpu.VMEM((B, D), f32)] * 2,
          [pltpu.VMEM((B,), f32)] * 2,
          [pltpu.VMEM((B,), f32)] * 2,
          pltpu.VMEM((16 * D,), f32),
          pltpu.SemaphoreType.DMA,
          pltpu.SemaphoreType.DMA,
          pltpu.SemaphoreType.DMA,
      ],
  )
  def _sc_edges(skn_hbm, g1_hbm, ak_hbm, bk_hbm, ahk_hbm, bhk_hbm, tbl_hbm,
                kv_hbm, zk_hbm, zh_hbm, outk_hbm, outh_hbm,
                acck, acch, tbl_v, kv, aidx, bidx, xa, xb, g1a, g1b,
                msga, msgb, fha, fhb, prods, isem, gsem, ssem):
    cid = lax.axis_index("c")
    sid = lax.axis_index("s")
    wid = sid * NC + cid

    pltpu.sync_copy(tbl_hbm, tbl_v)
    pltpu.sync_copy(kv_hbm, kv)
    r0 = sid * ROWS
    pltpu.sync_copy(zk_hbm.at[pl.ds(r0, ROWS)], acck.at[pl.ds(r0, ROWS)])
    pltpu.sync_copy(zh_hbm.at[pl.ds(r0, ROWS)], acch.at[pl.ds(r0, ROWS)])
    plsc.subcore_barrier()

    lane = lax.iota(i32, 16)

    def group_dot(xa_w, xb_w, j):
      e0 = j * 16
      va = []
      vb = []
      for e in range(16):
        a_row = xa_w[e0 + e, :]
        b_row = xb_w[e0 + e, :]
        va.append(a_row)
        vb.append(b_row)
        prods[pl.ds(e * D, D)] = a_row * b_row
      s = plsc.load_gather(prods, [lane * D])
      for d in range(1, D):
        s = s + plsc.load_gather(prods, [lane * D + d])
      return va, vb, s

    def k_compute(p):
      def group(j, c2):
        va, vb, s = group_dot(xa[p], xb[p], j)
        q = jnp.clip((s + 1.0) * ((T - 1) * 0.5), 0.0, T - 1.0)
        ii = jnp.minimum(q.astype(i32), T - 2)
        v0 = plsc.load_gather(tbl_v, [ii])
        v1 = plsc.load_gather(tbl_v, [ii + 1])
        c = v0 + (q - ii.astype(f32)) * (v1 - v0)
        e0 = j * 16
        for e in range(16):
          ce = c[e]
          msga[p][e0 + e, :] = ce * vb[e]
          msgb[p][e0 + e, :] = ce * va[e]
        return c2

      lax.fori_loop(0, B // 16, group, 0)

    def hk_compute(p):
      def group(j, c2):
        va, vb, gram = group_dot(xa[p], xb[p], j)
        e0 = j * 16
        ga = g1a[p][pl.ds(e0, 16)]
        gb = g1b[p][pl.ds(e0, 16)]
        fha[p][pl.ds(e0, 16)] = gram * gb
        fhb[p][pl.ds(e0, 16)] = gram * ga
        negg = kv[...] * ga * gb
        for e in range(16):
          ge = negg[e]
          msga[p][e0 + e, :] = ge * vb[e]
          msgb[p][e0 + e, :] = ge * va[e]
        return c2

      lax.fori_loop(0, B // 16, group, 0)

    def run_phase(a2_hbm, b2_hbm, row0, nb, hk):
      compute = hk_compute if hk else k_compute

      def idx_load(i):
        slot = lax.rem(i, 4)
        pltpu.async_copy(a2_hbm.at[pl.ds(row0 + i, 1)],
                         aidx.at[pl.ds(slot, 1)], isem)
        pltpu.async_copy(b2_hbm.at[pl.ds(row0 + i, 1)],
                         bidx.at[pl.ds(slot, 1)], isem)

      def idx_wait():
        pltpu.make_async_copy(a2_hbm.at[pl.ds(row0, 1)],
                              aidx.at[pl.ds(0, 1)], isem).wait()
        pltpu.make_async_copy(b2_hbm.at[pl.ds(row0, 1)],
                              bidx.at[pl.ds(0, 1)], isem).wait()

      def gathers(i, p):
        slot = lax.rem(i, 4)
        pltpu.async_copy(skn_hbm.at[aidx.at[slot]], xa[p], gsem)
        pltpu.async_copy(skn_hbm.at[bidx.at[slot]], xb[p], gsem)
        if hk:
          pltpu.async_copy(g1_hbm.at[aidx.at[slot]], g1a[p], gsem)
          pltpu.async_copy(g1_hbm.at[bidx.at[slot]], g1b[p], gsem)

      def gathers_wait(p):
        pltpu.make_async_copy(skn_hbm.at[aidx.at[0]], xa[p], gsem).wait()
        pltpu.make_async_copy(skn_hbm.at[bidx.at[0]], xb[p], gsem).wait()
        if hk:
          pltpu.make_async_copy(g1_hbm.at[aidx.at[0]], g1a[p], gsem).wait()
          pltpu.make_async_copy(g1_hbm.at[bidx.at[0]], g1b[p], gsem).wait()

      def scatters(i, p):
        slot = lax.rem(i, 4)
        pltpu.async_copy(msga[p], acck.at[aidx.at[slot]], ssem, add=True)
        pltpu.async_copy(msgb[p], acck.at[bidx.at[slot]], ssem, add=True)
        if hk:
          pltpu.async_copy(fha[p], acch.at[aidx.at[slot]], ssem, add=True)
          pltpu.async_copy(fhb[p], acch.at[bidx.at[slot]], ssem, add=True)

      def scatters_wait(p):
        pltpu.make_async_copy(msga[p], acck.at[aidx.at[0]], ssem).wait()
        pltpu.make_async_copy(msgb[p], acck.at[bidx.at[0]], ssem).wait()
        if hk:
          pltpu.make_async_copy(fha[p], acch.at[aidx.at[0]], ssem).wait()
          pltpu.make_async_copy(fhb[p], acch.at[bidx.at[0]], ssem).wait()

      idx_load(0)
      idx_wait()
      gathers(0, 0)
      idx_load(1)

      nbp = nb // 2

      def body(t, carry):
        @pl.when(t >= 1)
        def _():
          scatters_wait(0)

        @pl.when(t < nbp - 1)
        def _():
          idx_load(2 * t + 2)

        idx_wait()
        gathers(2 * t + 1, 1)
        gathers_wait(0)
        compute(0)
        scatters(2 * t, 0)

        @pl.when(t >= 1)
        def _():
          scatters_wait(1)

        @pl.when(t < nbp - 1)
        def _():
          idx_load(2 * t + 3)
          idx_wait()
          gathers(2 * t + 2, 0)

        gathers_wait(1)
        compute(1)
        scatters(2 * t + 1, 1)
        return carry

      lax.fori_loop(0, nbp, body, 0)
      scatters_wait(0)
      scatters_wait(1)

    nbk = EKP // (NW * B)
    run_phase(ak_hbm, bk_hbm, wid * nbk, nbk, False)
    nbh = EHKP // (NW * B)
    run_phase(ahk_hbm, bhk_hbm, wid * nbh, nbh, True)

    plsc.subcore_barrier()
    pltpu.sync_copy(acck.at[pl.ds(r0, ROWS)],
                    outk_hbm.at[pl.ds(cid * NP + r0, ROWS)])
    pltpu.sync_copy(acch.at[pl.ds(r0, ROWS)],
                    outh_hbm.at[pl.ds(cid * NP + r0, ROWS)])

  return _sc_edges


_sc_edges = _make_sc_edges()


def kernel(t, state_H, state_K, ind_K, ind_HK, kappa_K, kappa_H, W1, b1, W2,
           b2, omega, w_hop):
  del t
  sh2 = state_H.reshape(N, 1).astype(f32)
  wh2 = w_hop.reshape(N, 1).astype(f32)
  kh = jnp.reshape(kappa_H, (1, 1)).astype(f32)
  kk = jnp.reshape(kappa_K, (1, 1)).astype(f32)

  skn, g1, fhb = _prep(sh2, state_K.astype(f32), wh2, kh)
  tbl = _tbl(W1.astype(f32), b1.reshape(1, H).astype(f32),
             W2.astype(f32), b2.reshape(1, 1).astype(f32)).reshape(T)

  kvec = jnp.broadcast_to(
      (-(kappa_H.astype(f32) ** 2) / kappa_K.astype(f32)).reshape(1), (16,))

  sknp = jnp.concatenate([skn, jnp.zeros((NP - N, D), f32)], axis=0)
  g1p = jnp.concatenate([g1.reshape(N), jnp.zeros((NP - N,), f32)], axis=0)

  pad_k = jnp.full((EKP - E_K,), N, i32)
  pad_hk = jnp.full((EHKP - E_HK,), N, i32)
  ak = jnp.concatenate([ind_K[:, 0].astype(i32), pad_k]).reshape(EKP // B, B)
  bk = jnp.concatenate([ind_K[:, 1].astype(i32), pad_k]).reshape(EKP // B, B)
  ahk = jnp.concatenate(
      [ind_HK[:, 0].astype(i32), pad_hk]).reshape(EHKP // B, B)
  bhk = jnp.concatenate(
      [ind_HK[:, 1].astype(i32), pad_hk]).reshape(EHKP // B, B)

  zk = jnp.zeros((NP, D), f32)
  zh = jnp.zeros((NP,), f32)

  outk, outh = _sc_edges(sknp, g1p, ak, bk, ahk, bhk, tbl, kvec, zk, zh)

  aka = outk[:N]
  akb = outk[NP:NP + N]
  aha = outh[:N].reshape(N, 1)
  ahb = outh[NP:NP + N].reshape(N, 1)

  fh, fk = _post(skn, aka, akb, aha, ahb, fhb, omega.astype(f32))
  return fh.reshape(N), fk


# speedup vs baseline: 31.4476x; 1.0000x over previous
import functools

import jax
import jax.numpy as jnp
from jax import lax
from jax.experimental import pallas as pl
from jax.experimental.pallas import tpu as pltpu
from jax.experimental.pallas import tpu_sc as plsc

N = 100000
D = 16
H = 32
E_K = 3200000
E_HK = 1600000

NC = 2
NS = 16
NW = NC * NS
B = 128
T = 1024

NP = 100096
ROWS = NP // NS
W = 2
EKP = 392 * W * B * NW
EHKP = 196 * W * B * NW

BN = 2000
GRID = N // BN

f32 = jnp.float32
i32 = jnp.int32


def _prep_body(sh_ref, sk_ref, wh_ref, kh_ref, skn_ref, g1_ref, fhb_ref):
  sh = sh_ref[...]
  g = jnp.tanh(sh)
  fhb_ref[...] = -sh + wh_ref[...] * g
  g1_ref[...] = g / kh_ref[0, 0]
  sk = sk_ref[...]
  nrm = lax.rsqrt(jnp.sum(sk * sk, axis=1, keepdims=True))
  skn_ref[...] = sk * nrm


_prep = pl.pallas_call(
    _prep_body,
    grid=(GRID,),
    in_specs=[
        pl.BlockSpec((BN, 1), lambda i: (i, 0)),
        pl.BlockSpec((BN, D), lambda i: (i, 0)),
        pl.BlockSpec((BN, 1), lambda i: (i, 0)),
        pl.BlockSpec((1, 1), lambda i: (0, 0)),
    ],
    out_specs=[
        pl.BlockSpec((BN, D), lambda i: (i, 0)),
        pl.BlockSpec((BN, 1), lambda i: (i, 0)),
        pl.BlockSpec((BN, 1), lambda i: (i, 0)),
    ],
    out_shape=[
        jax.ShapeDtypeStruct((N, D), f32),
        jax.ShapeDtypeStruct((N, 1), f32),
        jax.ShapeDtypeStruct((N, 1), f32),
    ],
)


def _tbl_body(w1_ref, b1_ref, w2_ref, b2_ref, tbl_ref):
  w1 = w1_ref[...]
  b1 = b1_ref[...]
  w2 = w2_ref[...]
  r = lax.broadcasted_iota(i32, (8, 128), 0)
  c = lax.broadcasted_iota(i32, (8, 128), 1)
  x = (r * 128 + c).astype(f32) * (2.0 / (T - 1)) - 1.0
  acc = jnp.full((8, 128), b2_ref[0, 0], f32)
  for h in range(H):
    acc = acc + w2[h, 0] * jnp.tanh(x * w1[0, h] + b1[0, h])
  tbl_ref[...] = acc


_tbl = pl.pallas_call(
    _tbl_body,
    out_shape=jax.ShapeDtypeStruct((8, 128), f32),
)


def _post_body(skn_ref, aka_ref, akb_ref, aha_ref, ahb_ref, fhb_ref, om_ref,
               fh_ref, fk_ref):
  skn = skn_ref[...]
  fk = aka_ref[...] + akb_ref[...]
  om = om_ref[...]
  a = (om - om.T) * 0.5
  fk_ref[...] = (-fk + skn * jnp.sum(skn * fk, axis=1, keepdims=True)
                 + jnp.dot(skn, a, preferred_element_type=f32))
  fh_ref[...] = fhb_ref[...] + aha_ref[...] + ahb_ref[...]


_post = pl.pallas_call(
    _post_body,
    grid=(GRID,),
    in_specs=[
        pl.BlockSpec((BN, D), lambda i: (i, 0)),
        pl.BlockSpec((BN, D), lambda i: (i, 0)),
        pl.BlockSpec((BN, D), lambda i: (i, 0)),
        pl.BlockSpec((BN, 1), lambda i: (i, 0)),
        pl.BlockSpec((BN, 1), lambda i: (i, 0)),
        pl.BlockSpec((BN, 1), lambda i: (i, 0)),
        pl.BlockSpec((D, D), lambda i: (0, 0)),
    ],
    out_specs=[
        pl.BlockSpec((BN, 1), lambda i: (i, 0)),
        pl.BlockSpec((BN, D), lambda i: (i, 0)),
    ],
    out_shape=[
        jax.ShapeDtypeStruct((N, 1), f32),
        jax.ShapeDtypeStruct((N, D), f32),
    ],
)


def _make_sc_edges():
  mesh = plsc.VectorSubcoreMesh(core_axis_name="c", subcore_axis_name="s")

  @functools.partial(
      pl.kernel,
      mesh=mesh,
      compiler_params=pltpu.CompilerParams(
          needs_layout_passes=False, use_tc_tiling_on_sc=False),
      out_type=[
          jax.ShapeDtypeStruct((NC * NP, D), f32),
          jax.ShapeDtypeStruct((NC * NP,), f32),
      ],
      scratch_types=[
          pltpu.VMEM_SHARED((NP, D), f32),
          pltpu.VMEM_SHARED((NP,), f32),
          pltpu.VMEM((T,), f32),
          pltpu.VMEM((16,), f32),
          pltpu.VMEM((8, B), i32),
          pltpu.VMEM((8, B), i32),
          [pltpu.VMEM((B, D), f32)] * 2,
          [pltpu.VMEM((B, D), f32)] * 2,
          [pltpu.VMEM((B,), f32)] * 2,
          [pltpu.VMEM((B,), f32)] * 2,
          [pltpu.VMEM((B, D), f32)] * 2,
          [pltpu.VMEM((B, D), f32)] * 2,
          [pltpu.VMEM((B,), f32)] * 2,
          [pltpu.VMEM((B,), f32)] * 2,
          pltpu.VMEM((16 * D,), f32),
          pltpu.SemaphoreType.DMA,
          pltpu.SemaphoreType.DMA,
          pltpu.SemaphoreType.DMA,
      ],
  )
  def _sc_edges(skn_hbm, g1_hbm, ak_hbm, bk_hbm, ahk_hbm, bhk_hbm, tbl_hbm,
                kv_hbm, zk_hbm, zh_hbm, outk_hbm, outh_hbm,
                acck, acch, tbl_v, kv, aidx, bidx, xa, xb, g1a, g1b,
                msga, msgb, fha, fhb, prods, isem, gsem, ssem):
    cid = lax.axis_index("c")
    sid = lax.axis_index("s")
    wid = sid * NC + cid

    pltpu.sync_copy(tbl_hbm, tbl_v)
    pltpu.sync_copy(kv_hbm, kv)
    r0 = sid * ROWS
    pltpu.sync_copy(zk_hbm.at[pl.ds(r0, ROWS)], acck.at[pl.ds(r0, ROWS)])
    pltpu.sync_copy(zh_hbm.at[pl.ds(r0, ROWS)], acch.at[pl.ds(r0, ROWS)])
    plsc.subcore_barrier()

    lane = lax.iota(i32, 16)

    def group_dot(xa_w, xb_w, j):
      e0 = j * 16
      va = []
      vb = []
      for e in range(16):
        a_row = xa_w[e0 + e, :]
        b_row = xb_w[e0 + e, :]
        va.append(a_row)
        vb.append(b_row)
        prods[pl.ds(e * D, D)] = a_row * b_row
      s = plsc.load_gather(prods, [lane * D])
      for d in range(1, D):
        s = s + plsc.load_gather(prods, [lane * D + d])
      return va, vb, s

    def k_compute(p):
      def group(j, c2):
        va, vb, s = group_dot(xa[p], xb[p], j)
        q = jnp.clip((s + 1.0) * ((T - 1) * 0.5), 0.0, T - 1.0)
        ii = jnp.minimum(q.astype(i32), T - 2)
        v0 = plsc.load_gather(tbl_v, [ii])
        v1 = plsc.load_gather(tbl_v, [ii + 1])
        c = v0 + (q - ii.astype(f32)) * (v1 - v0)
        e0 = j * 16
        for e in range(16):
          ce = c[e]
          msga[p][e0 + e, :] = ce * vb[e]
          msgb[p][e0 + e, :] = ce * va[e]
        return c2

      lax.fori_loop(0, B // 16, group, 0)

    def hk_compute(p):
      def group(j, c2):
        va, vb, gram = group_dot(xa[p], xb[p], j)
        e0 = j * 16
        ga = g1a[p][pl.ds(e0, 16)]
        gb = g1b[p][pl.ds(e0, 16)]
        fha[p][pl.ds(e0, 16)] = gram * gb
        fhb[p][pl.ds(e0, 16)] = gram * ga
        negg = kv[...] * ga * gb
        for e in range(16):
          ge = negg[e]
          msga[p][e0 + e, :] = ge * vb[e]
          msgb[p][e0 + e, :] = ge * va[e]
        return c2

      lax.fori_loop(0, B // 16, group, 0)

    def run_phase(a2_hbm, b2_hbm, row0, nb, hk):
      compute = hk_compute if hk else k_compute

      def idx_load(i):
        slot = lax.rem(i, 8)
        pltpu.async_copy(a2_hbm.at[pl.ds(row0 + i, 1)],
                         aidx.at[pl.ds(slot, 1)], isem)
        pltpu.async_copy(b2_hbm.at[pl.ds(row0 + i, 1)],
                         bidx.at[pl.ds(slot, 1)], isem)

      def idx_wait():
        pltpu.make_async_copy(a2_hbm.at[pl.ds(row0, 1)],
                              aidx.at[pl.ds(0, 1)], isem).wait()
        pltpu.make_async_copy(b2_hbm.at[pl.ds(row0, 1)],
                              bidx.at[pl.ds(0, 1)], isem).wait()

      def gathers(i, p):
        slot = lax.rem(i, 8)
        pltpu.async_copy(skn_hbm.at[aidx.at[slot]], xa[p], gsem)
        pltpu.async_copy(skn_hbm.at[bidx.at[slot]], xb[p], gsem)
        if hk:
          pltpu.async_copy(g1_hbm.at[aidx.at[slot]], g1a[p], gsem)
          pltpu.async_copy(g1_hbm.at[bidx.at[slot]], g1b[p], gsem)

      def gathers_wait(p):
        pltpu.make_async_copy(skn_hbm.at[aidx.at[0]], xa[p], gsem).wait()
        pltpu.make_async_copy(skn_hbm.at[bidx.at[0]], xb[p], gsem).wait()
        if hk:
          pltpu.make_async_copy(g1_hbm.at[aidx.at[0]], g1a[p], gsem).wait()
          pltpu.make_async_copy(g1_hbm.at[bidx.at[0]], g1b[p], gsem).wait()

      def scatters(i, p):
        slot = lax.rem(i, 8)
        pltpu.async_copy(msga[p], acck.at[aidx.at[slot]], ssem, add=True)
        pltpu.async_copy(msgb[p], acck.at[bidx.at[slot]], ssem, add=True)
        if hk:
          pltpu.async_copy(fha[p], acch.at[aidx.at[slot]], ssem, add=True)
          pltpu.async_copy(fhb[p], acch.at[bidx.at[slot]], ssem, add=True)

      def scatters_wait(p):
        pltpu.make_async_copy(msga[p], acck.at[aidx.at[0]], ssem).wait()
        pltpu.make_async_copy(msgb[p], acck.at[bidx.at[0]], ssem).wait()
        if hk:
          pltpu.make_async_copy(fha[p], acch.at[aidx.at[0]], ssem).wait()
          pltpu.make_async_copy(fhb[p], acch.at[bidx.at[0]], ssem).wait()

      idx_load(0)
      idx_wait()
      gathers(0, 0)
      idx_load(1)

      nbp = nb // 2

      def body(t, carry):
        @pl.when(t >= 1)
        def _():
          scatters_wait(0)

        @pl.when(t < nbp - 1)
        def _():
          idx_load(2 * t + 2)
          idx_load(2 * t + 3)

        idx_wait()
        gathers(2 * t + 1, 1)
        gathers_wait(0)
        compute(0)
        scatters(2 * t, 0)

        @pl.when(t >= 1)
        def _():
          scatters_wait(1)

        @pl.when(t < nbp - 1)
        def _():
          idx_wait()
          gathers(2 * t + 2, 0)

        gathers_wait(1)
        compute(1)
        scatters(2 * t + 1, 1)
        return carry

      lax.fori_loop(0, nbp, body, 0)
      scatters_wait(0)
      scatters_wait(1)

    nbk = EKP // (NW * B)
    run_phase(ak_hbm, bk_hbm, wid * nbk, nbk, False)
    nbh = EHKP // (NW * B)
    run_phase(ahk_hbm, bhk_hbm, wid * nbh, nbh, True)

    plsc.subcore_barrier()
    pltpu.sync_copy(acck.at[pl.ds(r0, ROWS)],
                    outk_hbm.at[pl.ds(cid * NP + r0, ROWS)])
    pltpu.sync_copy(acch.at[pl.ds(r0, ROWS)],
                    outh_hbm.at[pl.ds(cid * NP + r0, ROWS)])

  return _sc_edges


_sc_edges = _make_sc_edges()


def kernel(t, state_H, state_K, ind_K, ind_HK, kappa_K, kappa_H, W1, b1, W2,
           b2, omega, w_hop):
  del t
  sh2 = state_H.reshape(N, 1).astype(f32)
  wh2 = w_hop.reshape(N, 1).astype(f32)
  kh = jnp.reshape(kappa_H, (1, 1)).astype(f32)
  kk = jnp.reshape(kappa_K, (1, 1)).astype(f32)

  skn, g1, fhb = _prep(sh2, state_K.astype(f32), wh2, kh)
  tbl = _tbl(W1.astype(f32), b1.reshape(1, H).astype(f32),
             W2.astype(f32), b2.reshape(1, 1).astype(f32)).reshape(T)

  kvec = jnp.broadcast_to(
      (-(kappa_H.astype(f32) ** 2) / kappa_K.astype(f32)).reshape(1), (16,))

  sknp = jnp.concatenate([skn, jnp.zeros((NP - N, D), f32)], axis=0)
  g1p = jnp.concatenate([g1.reshape(N), jnp.zeros((NP - N,), f32)], axis=0)

  pad_k = jnp.full((EKP - E_K,), N, i32)
  pad_hk = jnp.full((EHKP - E_HK,), N, i32)
  ak = jnp.concatenate([ind_K[:, 0].astype(i32), pad_k]).reshape(EKP // B, B)
  bk = jnp.concatenate([ind_K[:, 1].astype(i32), pad_k]).reshape(EKP // B, B)
  ahk = jnp.concatenate(
      [ind_HK[:, 0].astype(i32), pad_hk]).reshape(EHKP // B, B)
  bhk = jnp.concatenate(
      [ind_HK[:, 1].astype(i32), pad_hk]).reshape(EHKP // B, B)

  zk = jnp.zeros((NP, D), f32)
  zh = jnp.zeros((NP,), f32)

  outk, outh = _sc_edges(sknp, g1p, ak, bk, ahk, bhk, tbl, kvec, zk, zh)

  aka = outk[:N]
  akb = outk[NP:NP + N]
  aha = outh[:N].reshape(N, 1)
  ahb = outh[NP:NP + N].reshape(N, 1)

  fh, fk = _post(skn, aka, akb, aha, ahb, fhb, omega.astype(f32))
  return fh.reshape(N), fk


# speedup vs baseline: 35.5132x; 1.1293x over previous
import functools

import jax
import jax.numpy as jnp
from jax import lax
from jax.experimental import pallas as pl
from jax.experimental.pallas import tpu as pltpu
from jax.experimental.pallas import tpu_sc as plsc

N = 100000
D = 16
H = 32
E_K = 3200000
E_HK = 1600000

NC = 2
NS = 16
NW = NC * NS
B = 128
T = 1024

NP = 100096
ROWS = NP // NS
W = 2
EKP = 392 * W * B * NW
EHKP = 196 * W * B * NW

BN = 5000
GRID = N // BN
NH = N // 16

f32 = jnp.float32
i32 = jnp.int32


def _prep_k_body(sk_ref, skn_ref):
  sk = sk_ref[...]
  nrm = lax.rsqrt(jnp.sum(sk * sk, axis=1, keepdims=True))
  skn_ref[...] = sk * nrm


_prep_k = pl.pallas_call(
    _prep_k_body,
    grid=(GRID,),
    in_specs=[pl.BlockSpec((BN, D), lambda i: (i, 0))],
    out_specs=pl.BlockSpec((BN, D), lambda i: (i, 0)),
    out_shape=jax.ShapeDtypeStruct((N, D), f32),
)


def _prep_h_body(sh_ref, wh_ref, kh_ref, g1_ref, fhb_ref):
  sh = sh_ref[...]
  g = jnp.tanh(sh)
  fhb_ref[...] = -sh + wh_ref[...] * g
  g1_ref[...] = g / kh_ref[0, 0]


_prep_h = pl.pallas_call(
    _prep_h_body,
    out_shape=[
        jax.ShapeDtypeStruct((NH, 16), f32),
        jax.ShapeDtypeStruct((NH, 16), f32),
    ],
)


def _tbl_body(w1_ref, b1_ref, w2_ref, b2_ref, tbl_ref):
  w1 = w1_ref[...]
  b1 = b1_ref[...]
  w2 = w2_ref[...]
  r = lax.broadcasted_iota(i32, (8, 128), 0)
  c = lax.broadcasted_iota(i32, (8, 128), 1)
  x = (r * 128 + c).astype(f32) * (2.0 / (T - 1)) - 1.0
  acc = jnp.full((8, 128), b2_ref[0, 0], f32)
  for h in range(H):
    acc = acc + w2[h, 0] * jnp.tanh(x * w1[0, h] + b1[0, h])
  tbl_ref[...] = acc


_tbl = pl.pallas_call(
    _tbl_body,
    out_shape=jax.ShapeDtypeStruct((8, 128), f32),
)


def _post_k_body(skn_ref, aka_ref, akb_ref, om_ref, fk_ref):
  skn = skn_ref[...]
  fk = aka_ref[...] + akb_ref[...]
  om = om_ref[...]
  a = (om - om.T) * 0.5
  fk_ref[...] = (-fk + skn * jnp.sum(skn * fk, axis=1, keepdims=True)
                 + jnp.dot(skn, a, preferred_element_type=f32))


_post_k = pl.pallas_call(
    _post_k_body,
    grid=(GRID,),
    in_specs=[
        pl.BlockSpec((BN, D), lambda i: (i, 0)),
        pl.BlockSpec((BN, D), lambda i: (i, 0)),
        pl.BlockSpec((BN, D), lambda i: (i, 0)),
        pl.BlockSpec((D, D), lambda i: (0, 0)),
    ],
    out_specs=pl.BlockSpec((BN, D), lambda i: (i, 0)),
    out_shape=jax.ShapeDtypeStruct((N, D), f32),
)


def _post_h_body(fhb_ref, aha_ref, ahb_ref, fh_ref):
  fh_ref[...] = fhb_ref[...] + aha_ref[...] + ahb_ref[...]


_post_h = pl.pallas_call(
    _post_h_body,
    out_shape=jax.ShapeDtypeStruct((NH, 16), f32),
)


def _make_sc_edges():
  mesh = plsc.VectorSubcoreMesh(core_axis_name="c", subcore_axis_name="s")

  @functools.partial(
      pl.kernel,
      mesh=mesh,
      compiler_params=pltpu.CompilerParams(
          needs_layout_passes=False, use_tc_tiling_on_sc=False),
      out_type=[
          jax.ShapeDtypeStruct((NC * NP, D), f32),
          jax.ShapeDtypeStruct((NC * NP,), f32),
      ],
      scratch_types=[
          pltpu.VMEM_SHARED((NP, D), f32),
          pltpu.VMEM_SHARED((NP,), f32),
          pltpu.VMEM((T,), f32),
          pltpu.VMEM((16,), f32),
          pltpu.VMEM((8, B), i32),
          pltpu.VMEM((8, B), i32),
          [pltpu.VMEM((B, D), f32)] * 2,
          [pltpu.VMEM((B, D), f32)] * 2,
          [pltpu.VMEM((B,), f32)] * 2,
          [pltpu.VMEM((B,), f32)] * 2,
          [pltpu.VMEM((B, D), f32)] * 2,
          [pltpu.VMEM((B, D), f32)] * 2,
          [pltpu.VMEM((B,), f32)] * 2,
          [pltpu.VMEM((B,), f32)] * 2,
          pltpu.VMEM((16 * D,), f32),
          pltpu.SemaphoreType.DMA,
          pltpu.SemaphoreType.DMA,
          pltpu.SemaphoreType.DMA,
      ],
  )
  def _sc_edges(skn_hbm, g1_hbm, ak_hbm, bk_hbm, ahk_hbm, bhk_hbm, tbl_hbm,
                kv_hbm, zk_hbm, zh_hbm, outk_hbm, outh_hbm,
                acck, acch, tbl_v, kv, aidx, bidx, xa, xb, g1a, g1b,
                msga, msgb, fha, fhb, prods, isem, gsem, ssem):
    cid = lax.axis_index("c")
    sid = lax.axis_index("s")
    wid = sid * NC + cid

    pltpu.sync_copy(tbl_hbm, tbl_v)
    pltpu.sync_copy(kv_hbm, kv)
    r0 = sid * ROWS
    pltpu.sync_copy(zk_hbm.at[pl.ds(r0, ROWS)], acck.at[pl.ds(r0, ROWS)])
    pltpu.sync_copy(zh_hbm.at[pl.ds(r0, ROWS)], acch.at[pl.ds(r0, ROWS)])
    plsc.subcore_barrier()

    lane = lax.iota(i32, 16)

    def group_dot(xa_w, xb_w, j):
      e0 = j * 16
      va = []
      vb = []
      for e in range(16):
        a_row = xa_w[e0 + e, :]
        b_row = xb_w[e0 + e, :]
        va.append(a_row)
        vb.append(b_row)
        prods[pl.ds(e * D, D)] = a_row * b_row
      s = plsc.load_gather(prods, [lane * D])
      for d in range(1, D):
        s = s + plsc.load_gather(prods, [lane * D + d])
      return va, vb, s

    def k_compute(p):
      def group(j, c2):
        va, vb, s = group_dot(xa[p], xb[p], j)
        q = jnp.clip((s + 1.0) * ((T - 1) * 0.5), 0.0, T - 1.0)
        ii = jnp.minimum(q.astype(i32), T - 2)
        v0 = plsc.load_gather(tbl_v, [ii])
        v1 = plsc.load_gather(tbl_v, [ii + 1])
        c = v0 + (q - ii.astype(f32)) * (v1 - v0)
        e0 = j * 16
        for e in range(16):
          ce = c[e]
          msga[p][e0 + e, :] = ce * vb[e]
          msgb[p][e0 + e, :] = ce * va[e]
        return c2

      lax.fori_loop(0, B // 16, group, 0)

    def hk_compute(p):
      def group(j, c2):
        va, vb, gram = group_dot(xa[p], xb[p], j)
        e0 = j * 16
        ga = g1a[p][pl.ds(e0, 16)]
        gb = g1b[p][pl.ds(e0, 16)]
        fha[p][pl.ds(e0, 16)] = gram * gb
        fhb[p][pl.ds(e0, 16)] = gram * ga
        negg = kv[...] * ga * gb
        for e in range(16):
          ge = negg[e]
          msga[p][e0 + e, :] = ge * vb[e]
          msgb[p][e0 + e, :] = ge * va[e]
        return c2

      lax.fori_loop(0, B // 16, group, 0)

    def run_phase(a2_hbm, b2_hbm, row0, nb, hk):
      compute = hk_compute if hk else k_compute

      def idx_load(i):
        slot = lax.rem(i, 8)
        pltpu.async_copy(a2_hbm.at[pl.ds(row0 + i, 1)],
                         aidx.at[pl.ds(slot, 1)], isem)
        pltpu.async_copy(b2_hbm.at[pl.ds(row0 + i, 1)],
                         bidx.at[pl.ds(slot, 1)], isem)

      def idx_wait():
        pltpu.make_async_copy(a2_hbm.at[pl.ds(row0, 1)],
                              aidx.at[pl.ds(0, 1)], isem).wait()
        pltpu.make_async_copy(b2_hbm.at[pl.ds(row0, 1)],
                              bidx.at[pl.ds(0, 1)], isem).wait()

      def gathers(i, p):
        slot = lax.rem(i, 8)
        pltpu.async_copy(skn_hbm.at[aidx.at[slot]], xa[p], gsem)
        pltpu.async_copy(skn_hbm.at[bidx.at[slot]], xb[p], gsem)
        if hk:
          pltpu.async_copy(g1_hbm.at[aidx.at[slot]], g1a[p], gsem)
          pltpu.async_copy(g1_hbm.at[bidx.at[slot]], g1b[p], gsem)

      def gathers_wait(p):
        pltpu.make_async_copy(skn_hbm.at[aidx.at[0]], xa[p], gsem).wait()
        pltpu.make_async_copy(skn_hbm.at[bidx.at[0]], xb[p], gsem).wait()
        if hk:
          pltpu.make_async_copy(g1_hbm.at[aidx.at[0]], g1a[p], gsem).wait()
          pltpu.make_async_copy(g1_hbm.at[bidx.at[0]], g1b[p], gsem).wait()

      def scatters(i, p):
        slot = lax.rem(i, 8)
        pltpu.async_copy(msga[p], acck.at[aidx.at[slot]], ssem, add=True)
        pltpu.async_copy(msgb[p], acck.at[bidx.at[slot]], ssem, add=True)
        if hk:
          pltpu.async_copy(fha[p], acch.at[aidx.at[slot]], ssem, add=True)
          pltpu.async_copy(fhb[p], acch.at[bidx.at[slot]], ssem, add=True)

      def scatters_wait(p):
        pltpu.make_async_copy(msga[p], acck.at[aidx.at[0]], ssem).wait()
        pltpu.make_async_copy(msgb[p], acck.at[bidx.at[0]], ssem).wait()
        if hk:
          pltpu.make_async_copy(fha[p], acch.at[aidx.at[0]], ssem).wait()
          pltpu.make_async_copy(fhb[p], acch.at[bidx.at[0]], ssem).wait()

      idx_load(0)
      idx_wait()
      gathers(0, 0)
      idx_load(1)

      nbp = nb // 2

      def body(t, carry):
        @pl.when(t >= 1)
        def _():
          scatters_wait(0)

        @pl.when(t < nbp - 1)
        def _():
          idx_load(2 * t + 2)
          idx_load(2 * t + 3)

        idx_wait()
        gathers(2 * t + 1, 1)
        gathers_wait(0)
        compute(0)
        scatters(2 * t, 0)

        @pl.when(t >= 1)
        def _():
          scatters_wait(1)

        @pl.when(t < nbp - 1)
        def _():
          idx_wait()
          gathers(2 * t + 2, 0)

        gathers_wait(1)
        compute(1)
        scatters(2 * t + 1, 1)
        return carry

      lax.fori_loop(0, nbp, body, 0)
      scatters_wait(0)
      scatters_wait(1)

    nbk = EKP // (NW * B)
    run_phase(ak_hbm, bk_hbm, wid * nbk, nbk, False)
    nbh = EHKP // (NW * B)
    run_phase(ahk_hbm, bhk_hbm, wid * nbh, nbh, True)

    plsc.subcore_barrier()
    pltpu.sync_copy(acck.at[pl.ds(r0, ROWS)],
                    outk_hbm.at[pl.ds(cid * NP + r0, ROWS)])
    pltpu.sync_copy(acch.at[pl.ds(r0, ROWS)],
                    outh_hbm.at[pl.ds(cid * NP + r0, ROWS)])

  return _sc_edges


_sc_edges = _make_sc_edges()


def kernel(t, state_H, state_K, ind_K, ind_HK, kappa_K, kappa_H, W1, b1, W2,
           b2, omega, w_hop):
  del t
  sh2 = state_H.reshape(NH, 16).astype(f32)
  wh2 = w_hop.reshape(NH, 16).astype(f32)
  kh = jnp.reshape(kappa_H, (1, 1)).astype(f32)

  skn = _prep_k(state_K.astype(f32))
  g1h, fhbh = _prep_h(sh2, wh2, kh)
  tbl = _tbl(W1.astype(f32), b1.reshape(1, H).astype(f32),
             W2.astype(f32), b2.reshape(1, 1).astype(f32)).reshape(T)

  kvec = jnp.broadcast_to(
      (-(kappa_H.astype(f32) ** 2) / kappa_K.astype(f32)).reshape(1), (16,))

  sknp = jnp.concatenate([skn, jnp.zeros((NP - N, D), f32)], axis=0)
  g1p = jnp.concatenate([g1h.reshape(N), jnp.zeros((NP - N,), f32)], axis=0)

  pad_k = jnp.full((EKP - E_K,), N, i32)
  pad_hk = jnp.full((EHKP - E_HK,), N, i32)
  ak = jnp.concatenate([ind_K[:, 0].astype(i32), pad_k]).reshape(EKP // B, B)
  bk = jnp.concatenate([ind_K[:, 1].astype(i32), pad_k]).reshape(EKP // B, B)
  ahk = jnp.concatenate(
      [ind_HK[:, 0].astype(i32), pad_hk]).reshape(EHKP // B, B)
  bhk = jnp.concatenate(
      [ind_HK[:, 1].astype(i32), pad_hk]).reshape(EHKP // B, B)

  zk = jnp.zeros((NP, D), f32)
  zh = jnp.zeros((NP,), f32)

  outk, outh = _sc_edges(sknp, g1p, ak, bk, ahk, bhk, tbl, kvec, zk, zh)

  aka = outk[:N]
  akb = outk[NP:NP + N]
  aha = outh[:N].reshape(NH, 16)
  ahb = outh[NP:NP + N].reshape(NH, 16)

  fk = _post_k(skn, aka, akb, omega.astype(f32))
  fh = _post_h(fhbh, aha, ahb)
  return fh.reshape(N), fk
